# Initial kernel scaffold; baseline (speedup 1.0000x reference)
import jax
import jax.numpy as jnp
from jax.experimental import pallas as pl


def kernel(inputs, w):
    raise NotImplementedError("write your pallas kernel here")


# speedup vs baseline: 73.4205x; 73.4205x over previous
import jax
import jax.numpy as jnp
from jax.experimental import pallas as pl

_UNITS = 128
_D = 256
_BB = 64


def _top2_body(x_ref, wt_ref, o_ref):
    m1 = jnp.full((_BB, _UNITS), -jnp.inf, dtype=jnp.float32)
    m2 = m1
    for d in range(_D):
        v = x_ref[:, d : d + 1] + wt_ref[d : d + 1, :]
        t = jnp.minimum(m1, v)
        m1 = jnp.maximum(m1, v)
        m2 = jnp.maximum(m2, t)
    o_ref[...] = m1 - m2


def kernel(inputs, w):
    wt = w.T
    batch = inputs.shape[0]
    return pl.pallas_call(
        _top2_body,
        grid=(batch // _BB,),
        in_specs=[
            pl.BlockSpec((_BB, _D), lambda i: (i, 0)),
            pl.BlockSpec((_D, _UNITS), lambda i: (0, 0)),
        ],
        out_specs=pl.BlockSpec((_BB, _UNITS), lambda i: (i, 0)),
        out_shape=jax.ShapeDtypeStruct((batch, _UNITS), jnp.float32),
    )(inputs, wt)
